# baseline (speedup 1.0000x reference)
import functools
import math

import jax
import jax.numpy as jnp
from jax import lax
from jax.experimental import pallas as pl
from jax.experimental.pallas import tpu as pltpu
from jax.experimental.pallas import tpu_sc as plsc

B = 4
C = 96
H = 160
W = 160
WIN = 5
PAD = WIN // 2
P = WIN * WIN
TEMP = 0.5
LAMBDA_SCALE = 0.5
K = 2560
N = H * W
ROWS = N // 128
SROWS = 256
NSORT = SROWS * 128
OUT_ROWS = 32


def _topk_body(sal_ref, msk_ref, cal_ref, idx_ref, row_ref, col_ref):
    z = (sal_ref[...] + msk_ref[...]) * (1.0 / TEMP)
    m = jnp.max(z)
    e = jnp.exp(z - m)
    cal = e / jnp.sum(e)
    cal_ref[...] = cal

    vals = jnp.concatenate(
        [cal, jnp.full((SROWS - ROWS, 128), -1.0, jnp.float32)], axis=0)
    r_iota = lax.broadcasted_iota(jnp.int32, (SROWS, 128), 0)
    c_iota = lax.broadcasted_iota(jnp.int32, (SROWS, 128), 1)
    flat_i = r_iota * 128 + c_iota
    idx = flat_i

    k = 2
    while k <= NSORT:
        j = k // 2
        while j >= 1:
            if j >= 128:
                d = j // 128
                bit = (r_iota & d) == 0
                pv = jnp.where(bit, pltpu.roll(vals, SROWS - d, 0), pltpu.roll(vals, d, 0))
                pi = jnp.where(bit, pltpu.roll(idx, SROWS - d, 0), pltpu.roll(idx, d, 0))
            else:
                bit = (c_iota & j) == 0
                pv = jnp.where(bit, pltpu.roll(vals, 128 - j, 1), pltpu.roll(vals, j, 1))
                pi = jnp.where(bit, pltpu.roll(idx, 128 - j, 1), pltpu.roll(idx, j, 1))
            mine_first = (vals > pv) | ((vals == pv) & (idx < pi))
            cmpmask = bit == ((flat_i & k) == 0)
            keep = mine_first == cmpmask
            vals = jnp.where(keep, vals, pv)
            idx = jnp.where(keep, idx, pi)
            j //= 2
        k *= 2

    top = idx[:OUT_ROWS, :]
    idx_ref[...] = top
    row_ref[...] = jnp.clip(top // W, PAD, H - 1 - PAD)
    col_ref[...] = jnp.clip(top % W, PAD, W - 1 - PAD)


def _run_topk(sal_flat, msk_flat, interpret=False):
    return pl.pallas_call(
        _topk_body,
        grid=(B,),
        in_specs=[
            pl.BlockSpec((None, ROWS, 128), lambda b: (b, 0, 0)),
            pl.BlockSpec((None, ROWS, 128), lambda b: (0, 0, 0)),
        ],
        out_specs=[
            pl.BlockSpec((None, ROWS, 128), lambda b: (b, 0, 0)),
            pl.BlockSpec((None, OUT_ROWS, 128), lambda b: (b, 0, 0)),
            pl.BlockSpec((None, OUT_ROWS, 128), lambda b: (b, 0, 0)),
            pl.BlockSpec((None, OUT_ROWS, 128), lambda b: (b, 0, 0)),
        ],
        out_shape=[
            jax.ShapeDtypeStruct((B, ROWS, 128), jnp.float32),
            jax.ShapeDtypeStruct((B, OUT_ROWS, 128), jnp.int32),
            jax.ShapeDtypeStruct((B, OUT_ROWS, 128), jnp.int32),
            jax.ShapeDtypeStruct((B, OUT_ROWS, 128), jnp.int32),
        ],
        interpret=interpret,
    )(sal_flat, msk_flat)


NC = 2
NS = 16
NWK = NC * NS
WPT = (B * K) // NWK
CH = 16
NCH = WPT // CH
CBLK = C // 16
CP = 128

_OFFS = [(dy, dx) for dy in range(-PAD, PAD + 1) for dx in range(-PAD, PAD + 1)]
_DW = [math.exp(-math.sqrt(dy * dy + dx * dx) / (LAMBDA_SCALE * WIN))
       for dy, dx in _OFFS]


def _sc_body(feat_hbm, cal_hbm, topk_hbm, gamma_hbm, out_hbm,
             score_v, topk_v, gamma_v, idx0, idx1, patch0, patch1,
             sbuf_v, mask_v, gsem0, gsem1, osem0, osem1):
    cid = lax.axis_index("c")
    sid = lax.axis_index("s")
    wid = sid * NC + cid
    w0 = wid * WPT
    b = w0 // K

    pltpu.sync_copy(cal_hbm.at[pl.ds(b * N, N)], score_v)
    pltpu.sync_copy(topk_hbm.at[pl.ds(w0, WPT)], topk_v)
    pltpu.sync_copy(gamma_hbm, gamma_v)
    gvec = gamma_v[...]
    lane = lax.iota(jnp.int32, 16)
    base_feat = b * N
    idxs = (idx0, idx1)
    patches = (patch0, patch1)
    gsems = (gsem0, gsem1)
    osems = (osem0, osem1)

    def win_coords(g):
        kid = topk_v[pl.ds(g * CH, CH)]
        r = jnp.clip(kid // W, PAD, H - 1 - PAD)
        c = jnp.clip(kid - (kid // W) * W, PAD, W - 1 - PAD)
        return r, c

    def build_idx(g, idx_v):
        r, c = win_coords(g)
        for p, (dy, dx) in enumerate(_OFFS):
            lin = (r + dy) * W + (c + dx)
            plsc.store_scatter(idx_v, [lane * P + p], base_feat + lin)

    def gather_copies(idx_v, patch_v, gs):
        return [
            pltpu.make_async_copy(
                feat_hbm.at[idx_v.at[pl.ds(jj * 80, 80)]],
                patch_v.at[pl.ds(jj * 80, 80)],
                gs,
            )
            for jj in range(5)
        ]

    def fire_gather(g, bb):
        build_idx(g, idxs[bb])
        for cp in gather_copies(idxs[bb], patches[bb], gsems[bb]):
            cp.start()

    def wait_gather(bb):
        for cp in gather_copies(idxs[bb], patches[bb], gsems[bb]):
            cp.wait()

    def mask_compute(g):
        r, c = win_coords(g)
        ssum = jnp.zeros((16,), jnp.float32)
        for p, (dy, dx) in enumerate(_OFFS):
            s = plsc.load_gather(score_v, [(r + dy) * W + (c + dx)])
            sbuf_v[p, :] = s
            ssum = ssum + s
        mean = ssum * (1.0 / P)
        for p in range(P):
            x = gvec * (sbuf_v[p, :] - mean)
            mask_v[pl.ds(p * 16, 16)] = _DW[p] / (1.0 + jnp.exp(-x))

    def multiply(bb):
        patch_v = patches[bb]

        @plsc.parallel_loop(0, CH, unroll=4)
        def mrow(w):
            for p in range(P):
                m = plsc.load_gather(mask_v, [lane * 0 + (p * 16 + w)])
                row = w * P + p
                for cb in range(CBLK):
                    sl = pl.ds(cb * 16, 16)
                    patch_v[row, sl] = patch_v[row, sl] * m

    def out_copy(g, bb):
        return pltpu.make_async_copy(
            patches[bb], out_hbm.at[pl.ds((w0 + g * CH) * P, CH * P)], osems[bb])

    def iteration(g, bb, tail, first):
        mask_compute(g)
        wait_gather(bb)
        if tail:
            nb = 1 - bb
            if not first:
                out_copy(g, nb).wait()
            fire_gather(g + 1, nb)
        multiply(bb)
        out_copy(g, bb).start()

    fire_gather(0, 0)
    iteration(0, 0, True, True)

    def pair(m, carry):
        iteration(2 * m + 1, 1, True, False)
        iteration(2 * m + 2, 0, True, False)
        return carry

    lax.fori_loop(0, (NCH - 2) // 2, pair, 0)
    iteration(NCH - 1, 1, False, False)
    out_copy(NCH - 2, 0).wait()
    out_copy(NCH - 1, 1).wait()


def _run_sc(feat_flat, cal_flat, topk_flat, gamma_vec):
    mesh = plsc.VectorSubcoreMesh(core_axis_name="c", subcore_axis_name="s")
    fn = functools.partial(
        pl.kernel,
        mesh=mesh,
        compiler_params=pltpu.CompilerParams(
            needs_layout_passes=False, use_tc_tiling_on_sc=False),
        out_type=jax.ShapeDtypeStruct((B * K * P, C), jnp.float32),
        scratch_types=[
            pltpu.VMEM((N,), jnp.float32),
            pltpu.VMEM((WPT,), jnp.int32),
            pltpu.VMEM((16,), jnp.float32),
            pltpu.VMEM((CH * P,), jnp.int32),
            pltpu.VMEM((CH * P,), jnp.int32),
            pltpu.VMEM((CH * P, C), jnp.float32),
            pltpu.VMEM((CH * P, C), jnp.float32),
            pltpu.VMEM((P, 16), jnp.float32),
            pltpu.VMEM((P * 16,), jnp.float32),
            pltpu.SemaphoreType.DMA,
            pltpu.SemaphoreType.DMA,
            pltpu.SemaphoreType.DMA,
            pltpu.SemaphoreType.DMA,
        ],
    )(_sc_body)
    return fn(feat_flat, cal_flat, topk_flat, gamma_vec)


KB = 256
KT = K // KB


def _relayout_body(x_ref, o_ref):
    x = x_ref[...]
    o_ref[...] = jnp.transpose(x, (1, 0)).reshape(P, C, KB)


def _run_relayout(x):
    return pl.pallas_call(
        _relayout_body,
        grid=(B, KT),
        in_specs=[pl.BlockSpec((None, None, KB, P * C), lambda b, t: (b, t, 0, 0))],
        out_specs=pl.BlockSpec((None, P, C, KB), lambda b, t: (b, 0, 0, t)),
        out_shape=jax.ShapeDtypeStruct((B, P, C, K), jnp.float32),
    )(x)


def _offsets_const():
    y = jnp.arange(-PAD, PAD + 1)
    x = jnp.arange(-PAD, PAD + 1)
    gy, gx = jnp.meshgrid(y, x, indexing='ij')
    return jnp.stack([gy.flatten(), gx.flatten()], axis=-1).reshape(1, 1, P, 2)


@jax.jit
def kernel(feat_map, saliency_map, mask_logits, gamma):
    sal_flat = saliency_map.reshape(B, ROWS, 128)
    msk_flat = mask_logits.reshape(1, ROWS, 128)
    cal, idxp, rowp, colp = _run_topk(sal_flat, msk_flat)

    calibrated_map = cal.reshape(B, H, W)
    topk_idx = idxp.reshape(B, OUT_ROWS * 128)[:, :K]
    rows = rowp.reshape(B, OUT_ROWS * 128)[:, :K]
    cols = colp.reshape(B, OUT_ROWS * 128)[:, :K]
    topk_coords = jnp.stack([rows, cols], axis=-1)

    feat_flat = jnp.transpose(feat_map, (0, 2, 3, 1)).reshape(B * N, C)
    cal_flat = cal.reshape(B * N)
    gamma_vec = jnp.broadcast_to(gamma.reshape(1), (16,)).astype(jnp.float32)

    out = _run_sc(feat_flat, cal_flat, topk_idx.reshape(B * K), gamma_vec)
    bpck = _run_relayout(out.reshape(B, KT, KB, P * C))
    patches = jnp.transpose(bpck, (0, 3, 1, 2))

    return (patches, topk_coords, _offsets_const(), calibrated_map)


# speedup vs baseline: 1.0266x; 1.0266x over previous
import functools
import math

import jax
import jax.numpy as jnp
from jax import lax
from jax.experimental import pallas as pl
from jax.experimental.pallas import tpu as pltpu
from jax.experimental.pallas import tpu_sc as plsc

B = 4
C = 96
H = 160
W = 160
WIN = 5
PAD = WIN // 2
P = WIN * WIN
TEMP = 0.5
LAMBDA_SCALE = 0.5
K = 2560
N = H * W
ROWS = N // 128
SROWS = 256
NSORT = SROWS * 128
OUT_ROWS = 32


def _topk_body(sal_ref, msk_ref, cal_ref, idx_ref, row_ref, col_ref):
    z = (sal_ref[...] + msk_ref[...]) * (1.0 / TEMP)
    m = jnp.max(z)
    e = jnp.exp(z - m)
    cal = e / jnp.sum(e)
    cal_ref[...] = cal

    vals = jnp.concatenate(
        [cal, jnp.full((SROWS - ROWS, 128), -1.0, jnp.float32)], axis=0)
    r_iota = lax.broadcasted_iota(jnp.int32, (SROWS, 128), 0)
    c_iota = lax.broadcasted_iota(jnp.int32, (SROWS, 128), 1)
    flat_i = r_iota * 128 + c_iota
    idx = flat_i

    k = 2
    while k <= NSORT:
        j = k // 2
        while j >= 1:
            if j >= 128:
                d = j // 128
                bit = (r_iota & d) == 0
                pv = jnp.where(bit, pltpu.roll(vals, SROWS - d, 0), pltpu.roll(vals, d, 0))
                pi = jnp.where(bit, pltpu.roll(idx, SROWS - d, 0), pltpu.roll(idx, d, 0))
            else:
                bit = (c_iota & j) == 0
                pv = jnp.where(bit, pltpu.roll(vals, 128 - j, 1), pltpu.roll(vals, j, 1))
                pi = jnp.where(bit, pltpu.roll(idx, 128 - j, 1), pltpu.roll(idx, j, 1))
            mine_first = (vals > pv) | ((vals == pv) & (idx < pi))
            cmpmask = bit == ((flat_i & k) == 0)
            keep = mine_first == cmpmask
            vals = jnp.where(keep, vals, pv)
            idx = jnp.where(keep, idx, pi)
            j //= 2
        k *= 2

    top = idx[:OUT_ROWS, :]
    idx_ref[...] = top
    row_ref[...] = jnp.clip(top // W, PAD, H - 1 - PAD)
    col_ref[...] = jnp.clip(top % W, PAD, W - 1 - PAD)


def _run_topk(sal_flat, msk_flat, interpret=False):
    return pl.pallas_call(
        _topk_body,
        grid=(B,),
        in_specs=[
            pl.BlockSpec((None, ROWS, 128), lambda b: (b, 0, 0)),
            pl.BlockSpec((None, ROWS, 128), lambda b: (0, 0, 0)),
        ],
        out_specs=[
            pl.BlockSpec((None, ROWS, 128), lambda b: (b, 0, 0)),
            pl.BlockSpec((None, OUT_ROWS, 128), lambda b: (b, 0, 0)),
            pl.BlockSpec((None, OUT_ROWS, 128), lambda b: (b, 0, 0)),
            pl.BlockSpec((None, OUT_ROWS, 128), lambda b: (b, 0, 0)),
        ],
        out_shape=[
            jax.ShapeDtypeStruct((B, ROWS, 128), jnp.float32),
            jax.ShapeDtypeStruct((B, OUT_ROWS, 128), jnp.int32),
            jax.ShapeDtypeStruct((B, OUT_ROWS, 128), jnp.int32),
            jax.ShapeDtypeStruct((B, OUT_ROWS, 128), jnp.int32),
        ],
        interpret=interpret,
    )(sal_flat, msk_flat)


NC = 2
NS = 16
NWK = NC * NS
WPT = (B * K) // NWK
CH = 16
NCH = WPT // CH
CBLK = C // 16
CP = 128

_OFFS = [(dy, dx) for dy in range(-PAD, PAD + 1) for dx in range(-PAD, PAD + 1)]
_DW = [math.exp(-math.sqrt(dy * dy + dx * dx) / (LAMBDA_SCALE * WIN))
       for dy, dx in _OFFS]


def _sc_body(feat_hbm, cal_hbm, topk_hbm, gamma_hbm, out_hbm,
             score_v, topk_v, gamma_v, idx0, idx1, patch0, patch1,
             sbuf_v, mask_v, gsem0, gsem1, osem0, osem1):
    cid = lax.axis_index("c")
    sid = lax.axis_index("s")
    wid = sid * NC + cid
    w0 = wid * WPT
    b = w0 // K

    pltpu.sync_copy(cal_hbm.at[pl.ds(b * N, N)], score_v)
    pltpu.sync_copy(topk_hbm.at[pl.ds(w0, WPT)], topk_v)
    pltpu.sync_copy(gamma_hbm, gamma_v)
    gvec = gamma_v[...]
    lane = lax.iota(jnp.int32, 16)
    base_feat = b * N
    idxs = (idx0, idx1)
    patches = (patch0, patch1)
    gsems = (gsem0, gsem1)
    osems = (osem0, osem1)

    def win_coords(g):
        kid = topk_v[pl.ds(g * CH, CH)]
        r = jnp.clip(kid // W, PAD, H - 1 - PAD)
        c = jnp.clip(kid - (kid // W) * W, PAD, W - 1 - PAD)
        return r, c

    def build_idx(g, idx_v):
        r, c = win_coords(g)
        for p, (dy, dx) in enumerate(_OFFS):
            lin = (r + dy) * W + (c + dx)
            plsc.store_scatter(idx_v, [lane * P + p], base_feat + lin)

    def gather_copies(idx_v, patch_v, gs):
        return [
            pltpu.make_async_copy(
                feat_hbm.at[idx_v.at[pl.ds(jj * 80, 80)]],
                patch_v.at[pl.ds(jj * 80, 80)],
                gs,
            )
            for jj in range(5)
        ]

    def fire_gather(g, bb):
        build_idx(g, idxs[bb])
        for cp in gather_copies(idxs[bb], patches[bb], gsems[bb]):
            cp.start()

    def wait_gather(bb):
        for cp in gather_copies(idxs[bb], patches[bb], gsems[bb]):
            cp.wait()

    def mask_compute(g):
        r, c = win_coords(g)
        ssum = jnp.zeros((16,), jnp.float32)
        for p, (dy, dx) in enumerate(_OFFS):
            s = plsc.load_gather(score_v, [(r + dy) * W + (c + dx)])
            sbuf_v[p, :] = s
            ssum = ssum + s
        mean = ssum * (1.0 / P)
        for p in range(P):
            x = gvec * (sbuf_v[p, :] - mean)
            mask_v[pl.ds(p * 16, 16)] = _DW[p] / (1.0 + jnp.exp(-x))

    def multiply(bb):
        patch_v = patches[bb]

        @plsc.parallel_loop(0, CH, unroll=2)
        def mrow(w):
            for p in range(P):
                m = plsc.load_gather(mask_v, [lane * 0 + (p * 16 + w)])
                row = w * P + p
                for cb in range(CBLK):
                    sl = pl.ds(cb * 16, 16)
                    patch_v[row, sl] = patch_v[row, sl] * m

    def out_copy(g, bb):
        return pltpu.make_async_copy(
            patches[bb], out_hbm.at[pl.ds((w0 + g * CH) * P, CH * P)], osems[bb])

    def iteration(g, bb, tail, first):
        mask_compute(g)
        wait_gather(bb)
        if tail:
            nb = 1 - bb
            if not first:
                out_copy(g, nb).wait()
            fire_gather(g + 1, nb)
        multiply(bb)
        out_copy(g, bb).start()

    fire_gather(0, 0)
    iteration(0, 0, True, True)

    def pair(m, carry):
        iteration(2 * m + 1, 1, True, False)
        iteration(2 * m + 2, 0, True, False)
        return carry

    lax.fori_loop(0, (NCH - 2) // 2, pair, 0)
    iteration(NCH - 1, 1, False, False)
    out_copy(NCH - 2, 0).wait()
    out_copy(NCH - 1, 1).wait()


def _run_sc(feat_flat, cal_flat, topk_flat, gamma_vec):
    mesh = plsc.VectorSubcoreMesh(core_axis_name="c", subcore_axis_name="s")
    fn = functools.partial(
        pl.kernel,
        mesh=mesh,
        compiler_params=pltpu.CompilerParams(
            needs_layout_passes=False, use_tc_tiling_on_sc=False),
        out_type=jax.ShapeDtypeStruct((B * K * P, C), jnp.float32),
        scratch_types=[
            pltpu.VMEM((N,), jnp.float32),
            pltpu.VMEM((WPT,), jnp.int32),
            pltpu.VMEM((16,), jnp.float32),
            pltpu.VMEM((CH * P,), jnp.int32),
            pltpu.VMEM((CH * P,), jnp.int32),
            pltpu.VMEM((CH * P, C), jnp.float32),
            pltpu.VMEM((CH * P, C), jnp.float32),
            pltpu.VMEM((P, 16), jnp.float32),
            pltpu.VMEM((P * 16,), jnp.float32),
            pltpu.SemaphoreType.DMA,
            pltpu.SemaphoreType.DMA,
            pltpu.SemaphoreType.DMA,
            pltpu.SemaphoreType.DMA,
        ],
    )(_sc_body)
    return fn(feat_flat, cal_flat, topk_flat, gamma_vec)


KB = 256
KT = K // KB


def _relayout_body(x_ref, o_ref):
    x = x_ref[...]
    o_ref[...] = jnp.transpose(x, (1, 0)).reshape(P, C, KB)


def _run_relayout(x):
    return pl.pallas_call(
        _relayout_body,
        grid=(B, KT),
        in_specs=[pl.BlockSpec((None, None, KB, P * C), lambda b, t: (b, t, 0, 0))],
        out_specs=pl.BlockSpec((None, P, C, KB), lambda b, t: (b, 0, 0, t)),
        out_shape=jax.ShapeDtypeStruct((B, P, C, K), jnp.float32),
    )(x)


def _offsets_const():
    y = jnp.arange(-PAD, PAD + 1)
    x = jnp.arange(-PAD, PAD + 1)
    gy, gx = jnp.meshgrid(y, x, indexing='ij')
    return jnp.stack([gy.flatten(), gx.flatten()], axis=-1).reshape(1, 1, P, 2)


@jax.jit
def kernel(feat_map, saliency_map, mask_logits, gamma):
    sal_flat = saliency_map.reshape(B, ROWS, 128)
    msk_flat = mask_logits.reshape(1, ROWS, 128)
    cal, idxp, rowp, colp = _run_topk(sal_flat, msk_flat)

    calibrated_map = cal.reshape(B, H, W)
    topk_idx = idxp.reshape(B, OUT_ROWS * 128)[:, :K]
    rows = rowp.reshape(B, OUT_ROWS * 128)[:, :K]
    cols = colp.reshape(B, OUT_ROWS * 128)[:, :K]
    topk_coords = jnp.stack([rows, cols], axis=-1)

    feat_flat = jnp.transpose(feat_map, (0, 2, 3, 1)).reshape(B * N, C)
    cal_flat = cal.reshape(B * N)
    gamma_vec = jnp.broadcast_to(gamma.reshape(1), (16,)).astype(jnp.float32)

    out = _run_sc(feat_flat, cal_flat, topk_idx.reshape(B * K), gamma_vec)
    bpck = _run_relayout(out.reshape(B, KT, KB, P * C))
    patches = jnp.transpose(bpck, (0, 3, 1, 2))

    return (patches, topk_coords, _offsets_const(), calibrated_map)


# speedup vs baseline: 1.0543x; 1.0270x over previous
import functools
import math

import jax
import jax.numpy as jnp
from jax import lax
from jax.experimental import pallas as pl
from jax.experimental.pallas import tpu as pltpu
from jax.experimental.pallas import tpu_sc as plsc

B = 4
C = 96
H = 160
W = 160
WIN = 5
PAD = WIN // 2
P = WIN * WIN
TEMP = 0.5
LAMBDA_SCALE = 0.5
K = 2560
N = H * W
ROWS = N // 128
SROWS = 256
NSORT = SROWS * 128
OUT_ROWS = 32


def _topk_body(sal_ref, msk_ref, cal_ref, idx_ref, row_ref, col_ref):
    z = (sal_ref[...] + msk_ref[...]) * (1.0 / TEMP)
    m = jnp.max(z)
    e = jnp.exp(z - m)
    cal = e / jnp.sum(e)
    cal_ref[...] = cal

    vals = jnp.concatenate(
        [cal, jnp.full((SROWS - ROWS, 128), -1.0, jnp.float32)], axis=0)
    r_iota = lax.broadcasted_iota(jnp.int32, (SROWS, 128), 0)
    c_iota = lax.broadcasted_iota(jnp.int32, (SROWS, 128), 1)
    idx = r_iota * 128 + c_iota

    CSZ = 4096
    CROWS = CSZ // 128

    def stage(vals, idx, j, kmask, nrows):
        ri = lax.broadcasted_iota(jnp.int32, (nrows, 128), 0)
        ci = lax.broadcasted_iota(jnp.int32, (nrows, 128), 1)
        fi = ri * 128 + ci
        if j >= 128:
            d = j // 128
            bit = (ri & d) == 0
            pv = jnp.where(bit, pltpu.roll(vals, nrows - d, 0), pltpu.roll(vals, d, 0))
            pi = jnp.where(bit, pltpu.roll(idx, nrows - d, 0), pltpu.roll(idx, d, 0))
        else:
            bit = (ci & j) == 0
            pv = jnp.where(bit, pltpu.roll(vals, 128 - j, 1), pltpu.roll(vals, j, 1))
            pi = jnp.where(bit, pltpu.roll(idx, 128 - j, 1), pltpu.roll(idx, j, 1))
        mine_first = (vals > pv) | ((vals == pv) & (idx < pi))
        cmpmask = bit == ((fi & kmask) == 0)
        keep = mine_first == cmpmask
        return jnp.where(keep, vals, pv), jnp.where(keep, idx, pi)

    k = 2
    while k <= CSZ:
        j = k // 2
        while j >= 1:
            vals, idx = stage(vals, idx, j, k, SROWS)
            j //= 2
        k *= 2

    nrows = SROWS
    while nrows > CROWS:
        vals, idx = stage(vals, idx, CSZ, CSZ, nrows)
        nrows //= 2
        g2 = 2 * CROWS
        vals = vals.reshape(nrows // CROWS, g2, 128)[:, :CROWS, :].reshape(nrows, 128)
        idx = idx.reshape(nrows // CROWS, g2, 128)[:, :CROWS, :].reshape(nrows, 128)
        j = CSZ // 2
        while j >= 1:
            vals, idx = stage(vals, idx, j, 2 * CSZ if nrows == CROWS else CSZ, nrows)
            j //= 2

    top = idx[:OUT_ROWS, :]
    idx_ref[...] = top
    row_ref[...] = jnp.clip(top // W, PAD, H - 1 - PAD)
    col_ref[...] = jnp.clip(top % W, PAD, W - 1 - PAD)


def _run_topk(sal_flat, msk_flat, interpret=False):
    return pl.pallas_call(
        _topk_body,
        grid=(B,),
        in_specs=[
            pl.BlockSpec((None, ROWS, 128), lambda b: (b, 0, 0)),
            pl.BlockSpec((None, ROWS, 128), lambda b: (0, 0, 0)),
        ],
        out_specs=[
            pl.BlockSpec((None, ROWS, 128), lambda b: (b, 0, 0)),
            pl.BlockSpec((None, OUT_ROWS, 128), lambda b: (b, 0, 0)),
            pl.BlockSpec((None, OUT_ROWS, 128), lambda b: (b, 0, 0)),
            pl.BlockSpec((None, OUT_ROWS, 128), lambda b: (b, 0, 0)),
        ],
        out_shape=[
            jax.ShapeDtypeStruct((B, ROWS, 128), jnp.float32),
            jax.ShapeDtypeStruct((B, OUT_ROWS, 128), jnp.int32),
            jax.ShapeDtypeStruct((B, OUT_ROWS, 128), jnp.int32),
            jax.ShapeDtypeStruct((B, OUT_ROWS, 128), jnp.int32),
        ],
        interpret=interpret,
    )(sal_flat, msk_flat)


NC = 2
NS = 16
NWK = NC * NS
WPT = (B * K) // NWK
CH = 16
NCH = WPT // CH
CBLK = C // 16
CP = 128

_OFFS = [(dy, dx) for dy in range(-PAD, PAD + 1) for dx in range(-PAD, PAD + 1)]
_DW = [math.exp(-math.sqrt(dy * dy + dx * dx) / (LAMBDA_SCALE * WIN))
       for dy, dx in _OFFS]


def _sc_body(feat_hbm, cal_hbm, topk_hbm, gamma_hbm, out_hbm,
             score_v, topk_v, gamma_v, idx0, idx1, patch0, patch1,
             sbuf_v, mask_v, gsem0, gsem1, osem0, osem1):
    cid = lax.axis_index("c")
    sid = lax.axis_index("s")
    wid = sid * NC + cid
    w0 = wid * WPT
    b = w0 // K

    pltpu.sync_copy(cal_hbm.at[pl.ds(b * N, N)], score_v)
    pltpu.sync_copy(topk_hbm.at[pl.ds(w0, WPT)], topk_v)
    pltpu.sync_copy(gamma_hbm, gamma_v)
    gvec = gamma_v[...]
    lane = lax.iota(jnp.int32, 16)
    base_feat = b * N
    idxs = (idx0, idx1)
    patches = (patch0, patch1)
    gsems = (gsem0, gsem1)
    osems = (osem0, osem1)

    def win_coords(g):
        kid = topk_v[pl.ds(g * CH, CH)]
        r = jnp.clip(kid // W, PAD, H - 1 - PAD)
        c = jnp.clip(kid - (kid // W) * W, PAD, W - 1 - PAD)
        return r, c

    def build_idx(g, idx_v):
        r, c = win_coords(g)
        for p, (dy, dx) in enumerate(_OFFS):
            lin = (r + dy) * W + (c + dx)
            plsc.store_scatter(idx_v, [lane * P + p], base_feat + lin)

    def gather_copies(idx_v, patch_v, gs):
        return [
            pltpu.make_async_copy(
                feat_hbm.at[idx_v.at[pl.ds(jj * 80, 80)]],
                patch_v.at[pl.ds(jj * 80, 80)],
                gs,
            )
            for jj in range(5)
        ]

    def fire_gather(g, bb):
        build_idx(g, idxs[bb])
        for cp in gather_copies(idxs[bb], patches[bb], gsems[bb]):
            cp.start()

    def wait_gather(bb):
        for cp in gather_copies(idxs[bb], patches[bb], gsems[bb]):
            cp.wait()

    def mask_compute(g):
        r, c = win_coords(g)
        ssum = jnp.zeros((16,), jnp.float32)
        for p, (dy, dx) in enumerate(_OFFS):
            s = plsc.load_gather(score_v, [(r + dy) * W + (c + dx)])
            sbuf_v[p, :] = s
            ssum = ssum + s
        mean = ssum * (1.0 / P)
        for p in range(P):
            x = gvec * (sbuf_v[p, :] - mean)
            mask_v[pl.ds(p * 16, 16)] = _DW[p] / (1.0 + jnp.exp(-x))

    def multiply(bb):
        patch_v = patches[bb]

        @plsc.parallel_loop(0, CH, unroll=2)
        def mrow(w):
            for p in range(P):
                m = plsc.load_gather(mask_v, [lane * 0 + (p * 16 + w)])
                row = w * P + p
                for cb in range(CBLK):
                    sl = pl.ds(cb * 16, 16)
                    patch_v[row, sl] = patch_v[row, sl] * m

    def out_copy(g, bb):
        return pltpu.make_async_copy(
            patches[bb], out_hbm.at[pl.ds((w0 + g * CH) * P, CH * P)], osems[bb])

    def iteration(g, bb, tail, first):
        mask_compute(g)
        wait_gather(bb)
        if tail:
            nb = 1 - bb
            if not first:
                out_copy(g, nb).wait()
            fire_gather(g + 1, nb)
        multiply(bb)
        out_copy(g, bb).start()

    fire_gather(0, 0)
    iteration(0, 0, True, True)

    def pair(m, carry):
        iteration(2 * m + 1, 1, True, False)
        iteration(2 * m + 2, 0, True, False)
        return carry

    lax.fori_loop(0, (NCH - 2) // 2, pair, 0)
    iteration(NCH - 1, 1, False, False)
    out_copy(NCH - 2, 0).wait()
    out_copy(NCH - 1, 1).wait()


def _run_sc(feat_flat, cal_flat, topk_flat, gamma_vec):
    mesh = plsc.VectorSubcoreMesh(core_axis_name="c", subcore_axis_name="s")
    fn = functools.partial(
        pl.kernel,
        mesh=mesh,
        compiler_params=pltpu.CompilerParams(
            needs_layout_passes=False, use_tc_tiling_on_sc=False),
        out_type=jax.ShapeDtypeStruct((B * K * P, C), jnp.float32),
        scratch_types=[
            pltpu.VMEM((N,), jnp.float32),
            pltpu.VMEM((WPT,), jnp.int32),
            pltpu.VMEM((16,), jnp.float32),
            pltpu.VMEM((CH * P,), jnp.int32),
            pltpu.VMEM((CH * P,), jnp.int32),
            pltpu.VMEM((CH * P, C), jnp.float32),
            pltpu.VMEM((CH * P, C), jnp.float32),
            pltpu.VMEM((P, 16), jnp.float32),
            pltpu.VMEM((P * 16,), jnp.float32),
            pltpu.SemaphoreType.DMA,
            pltpu.SemaphoreType.DMA,
            pltpu.SemaphoreType.DMA,
            pltpu.SemaphoreType.DMA,
        ],
    )(_sc_body)
    return fn(feat_flat, cal_flat, topk_flat, gamma_vec)


KB = 256
KT = K // KB


def _relayout_body(x_ref, o_ref):
    x = x_ref[...]
    o_ref[...] = jnp.transpose(x, (1, 0)).reshape(P, C, KB)


def _run_relayout(x):
    return pl.pallas_call(
        _relayout_body,
        grid=(B, KT),
        in_specs=[pl.BlockSpec((None, None, KB, P * C), lambda b, t: (b, t, 0, 0))],
        out_specs=pl.BlockSpec((None, P, C, KB), lambda b, t: (b, 0, 0, t)),
        out_shape=jax.ShapeDtypeStruct((B, P, C, K), jnp.float32),
    )(x)


def _offsets_const():
    y = jnp.arange(-PAD, PAD + 1)
    x = jnp.arange(-PAD, PAD + 1)
    gy, gx = jnp.meshgrid(y, x, indexing='ij')
    return jnp.stack([gy.flatten(), gx.flatten()], axis=-1).reshape(1, 1, P, 2)


@jax.jit
def kernel(feat_map, saliency_map, mask_logits, gamma):
    sal_flat = saliency_map.reshape(B, ROWS, 128)
    msk_flat = mask_logits.reshape(1, ROWS, 128)
    cal, idxp, rowp, colp = _run_topk(sal_flat, msk_flat)

    calibrated_map = cal.reshape(B, H, W)
    topk_idx = idxp.reshape(B, OUT_ROWS * 128)[:, :K]
    rows = rowp.reshape(B, OUT_ROWS * 128)[:, :K]
    cols = colp.reshape(B, OUT_ROWS * 128)[:, :K]
    topk_coords = jnp.stack([rows, cols], axis=-1)

    feat_flat = jnp.transpose(feat_map, (0, 2, 3, 1)).reshape(B * N, C)
    cal_flat = cal.reshape(B * N)
    gamma_vec = jnp.broadcast_to(gamma.reshape(1), (16,)).astype(jnp.float32)

    out = _run_sc(feat_flat, cal_flat, topk_idx.reshape(B * K), gamma_vec)
    bpck = _run_relayout(out.reshape(B, KT, KB, P * C))
    patches = jnp.transpose(bpck, (0, 3, 1, 2))

    return (patches, topk_coords, _offsets_const(), calibrated_map)


# speedup vs baseline: 1.0548x; 1.0005x over previous
import functools
import math

import jax
import jax.numpy as jnp
from jax import lax
from jax.experimental import pallas as pl
from jax.experimental.pallas import tpu as pltpu
from jax.experimental.pallas import tpu_sc as plsc

B = 4
C = 96
H = 160
W = 160
WIN = 5
PAD = WIN // 2
P = WIN * WIN
TEMP = 0.5
LAMBDA_SCALE = 0.5
K = 2560
N = H * W
ROWS = N // 128
SROWS = 256
NSORT = SROWS * 128
OUT_ROWS = 32


def _topk_body(sal_ref, msk_ref, cal_ref, idx_ref, row_ref, col_ref):
    z = (sal_ref[...] + msk_ref[...]) * (1.0 / TEMP)
    m = jnp.max(z)
    e = jnp.exp(z - m)
    cal = e / jnp.sum(e)
    cal_ref[...] = cal

    vals = jnp.concatenate(
        [cal, jnp.full((SROWS - ROWS, 128), -1.0, jnp.float32)], axis=0)
    r_iota = lax.broadcasted_iota(jnp.int32, (SROWS, 128), 0)
    c_iota = lax.broadcasted_iota(jnp.int32, (SROWS, 128), 1)
    idx = r_iota * 128 + c_iota

    CSZ = 4096
    CROWS = CSZ // 128

    def stage(vals, idx, j, kmask, nrows):
        ri = lax.broadcasted_iota(jnp.int32, (nrows, 128), 0)
        ci = lax.broadcasted_iota(jnp.int32, (nrows, 128), 1)
        fi = ri * 128 + ci
        if j >= 128:
            d = j // 128
            bit = (ri & d) == 0
            pv = jnp.where(bit, pltpu.roll(vals, nrows - d, 0), pltpu.roll(vals, d, 0))
            pi = jnp.where(bit, pltpu.roll(idx, nrows - d, 0), pltpu.roll(idx, d, 0))
        else:
            bit = (ci & j) == 0
            pv = jnp.where(bit, pltpu.roll(vals, 128 - j, 1), pltpu.roll(vals, j, 1))
            pi = jnp.where(bit, pltpu.roll(idx, 128 - j, 1), pltpu.roll(idx, j, 1))
        mine_first = (vals > pv) | ((vals == pv) & (idx < pi))
        cmpmask = bit == ((fi & kmask) == 0)
        keep = mine_first == cmpmask
        return jnp.where(keep, vals, pv), jnp.where(keep, idx, pi)

    k = 2
    while k <= CSZ:
        j = k // 2
        while j >= 1:
            vals, idx = stage(vals, idx, j, k, SROWS)
            j //= 2
        k *= 2

    nrows = SROWS
    while nrows > CROWS:
        vals, idx = stage(vals, idx, CSZ, CSZ, nrows)
        nrows //= 2
        g2 = 2 * CROWS
        vals = vals.reshape(nrows // CROWS, g2, 128)[:, :CROWS, :].reshape(nrows, 128)
        idx = idx.reshape(nrows // CROWS, g2, 128)[:, :CROWS, :].reshape(nrows, 128)
        j = CSZ // 2
        while j >= 1:
            vals, idx = stage(vals, idx, j, 2 * CSZ if nrows == CROWS else CSZ, nrows)
            j //= 2

    top = idx[:OUT_ROWS, :]
    idx_ref[...] = top
    row_ref[...] = jnp.clip(top // W, PAD, H - 1 - PAD)
    col_ref[...] = jnp.clip(top % W, PAD, W - 1 - PAD)


def _run_topk(sal_flat, msk_flat, interpret=False):
    return pl.pallas_call(
        _topk_body,
        grid=(B,),
        in_specs=[
            pl.BlockSpec((None, ROWS, 128), lambda b: (b, 0, 0)),
            pl.BlockSpec((None, ROWS, 128), lambda b: (0, 0, 0)),
        ],
        out_specs=[
            pl.BlockSpec((None, ROWS, 128), lambda b: (b, 0, 0)),
            pl.BlockSpec((None, OUT_ROWS, 128), lambda b: (b, 0, 0)),
            pl.BlockSpec((None, OUT_ROWS, 128), lambda b: (b, 0, 0)),
            pl.BlockSpec((None, OUT_ROWS, 128), lambda b: (b, 0, 0)),
        ],
        out_shape=[
            jax.ShapeDtypeStruct((B, ROWS, 128), jnp.float32),
            jax.ShapeDtypeStruct((B, OUT_ROWS, 128), jnp.int32),
            jax.ShapeDtypeStruct((B, OUT_ROWS, 128), jnp.int32),
            jax.ShapeDtypeStruct((B, OUT_ROWS, 128), jnp.int32),
        ],
        interpret=interpret,
    )(sal_flat, msk_flat)


NC = 2
NS = 16
NWK = NC * NS
WPT = (B * K) // NWK
CH = 16
NCH = WPT // CH
CBLK = C // 16

_OFFS = [(dy, dx) for dy in range(-PAD, PAD + 1) for dx in range(-PAD, PAD + 1)]
_DW = [math.exp(-math.sqrt(dy * dy + dx * dx) / (LAMBDA_SCALE * WIN))
       for dy, dx in _OFFS]


def _sc_body(feat_hbm, cal_hbm, topk_hbm, gamma_hbm, out_hbm,
             score_v, topk_v, gamma_v, idx0, idx1, patch0, patch1,
             sbuf_v, mask_v, gsem0, gsem1, osem0, osem1):
    cid = lax.axis_index("c")
    sid = lax.axis_index("s")
    wid = sid * NC + cid
    w0 = wid * WPT
    b = w0 // K

    pltpu.sync_copy(cal_hbm.at[pl.ds(b * N, N)], score_v)
    pltpu.sync_copy(topk_hbm.at[pl.ds(w0, WPT)], topk_v)
    pltpu.sync_copy(gamma_hbm, gamma_v)
    gvec = gamma_v[...]
    lane = lax.iota(jnp.int32, 16)
    base_feat = b * N
    idxs = (idx0, idx1)
    patches = (patch0, patch1)
    gsems = (gsem0, gsem1)
    osems = (osem0, osem1)

    def win_coords(g):
        kid = topk_v[pl.ds(g * CH, CH)]
        r = jnp.clip(kid // W, PAD, H - 1 - PAD)
        c = jnp.clip(kid - (kid // W) * W, PAD, W - 1 - PAD)
        return r, c

    def build_idx(g, idx_v):
        r, c = win_coords(g)
        for p, (dy, dx) in enumerate(_OFFS):
            lin = (r + dy) * W + (c + dx)
            plsc.store_scatter(idx_v, [lane * P + p], base_feat + lin)

    def gather_copies(idx_v, patch_v, gs):
        return [
            pltpu.make_async_copy(
                feat_hbm.at[idx_v.at[pl.ds(jj * 80, 80)]],
                patch_v.at[pl.ds(jj * 80, 80)],
                gs,
            )
            for jj in range(5)
        ]

    def fire_gather(g, bb):
        build_idx(g, idxs[bb])
        for cp in gather_copies(idxs[bb], patches[bb], gsems[bb]):
            cp.start()

    def wait_gather(bb):
        for cp in gather_copies(idxs[bb], patches[bb], gsems[bb]):
            cp.wait()

    def mask_compute(g):
        r, c = win_coords(g)
        ssum = jnp.zeros((16,), jnp.float32)
        for p, (dy, dx) in enumerate(_OFFS):
            s = plsc.load_gather(score_v, [(r + dy) * W + (c + dx)])
            sbuf_v[p, :] = s
            ssum = ssum + s
        mean = ssum * (1.0 / P)
        for p in range(P):
            x = gvec * (sbuf_v[p, :] - mean)
            mask_v[pl.ds(p * 16, 16)] = _DW[p] / (1.0 + jnp.exp(-x))

    def multiply(bb):
        patch_v = patches[bb]

        @plsc.parallel_loop(0, CH, unroll=2)
        def mrow(w):
            for p in range(P):
                m = plsc.load_gather(mask_v, [lane * 0 + (p * 16 + w)])
                row = w * P + p
                for cb in range(CBLK):
                    sl = pl.ds(cb * 16, 16)
                    patch_v[row, sl] = patch_v[row, sl] * m

    def out_copy(g, bb):
        return pltpu.make_async_copy(
            patches[bb], out_hbm.at[pl.ds((w0 + g * CH) * P, CH * P)], osems[bb])

    def iteration(g, bb, tail, first):
        mask_compute(g)
        wait_gather(bb)
        if tail:
            nb = 1 - bb
            if not first:
                out_copy(g, nb).wait()
            fire_gather(g + 1, nb)
        multiply(bb)
        out_copy(g, bb).start()

    fire_gather(0, 0)
    iteration(0, 0, True, True)

    def pair(m, carry):
        iteration(2 * m + 1, 1, True, False)
        iteration(2 * m + 2, 0, True, False)
        return carry

    lax.fori_loop(0, (NCH - 2) // 2, pair, 0)
    iteration(NCH - 1, 1, False, False)
    out_copy(NCH - 2, 0).wait()
    out_copy(NCH - 1, 1).wait()


def _run_sc(feat_flat, cal_flat, topk_flat, gamma_vec):
    mesh = plsc.VectorSubcoreMesh(core_axis_name="c", subcore_axis_name="s")
    fn = functools.partial(
        pl.kernel,
        mesh=mesh,
        compiler_params=pltpu.CompilerParams(
            needs_layout_passes=False, use_tc_tiling_on_sc=False),
        out_type=jax.ShapeDtypeStruct((B * K * P, C), jnp.float32),
        scratch_types=[
            pltpu.VMEM((N,), jnp.float32),
            pltpu.VMEM((WPT,), jnp.int32),
            pltpu.VMEM((16,), jnp.float32),
            pltpu.VMEM((CH * P,), jnp.int32),
            pltpu.VMEM((CH * P,), jnp.int32),
            pltpu.VMEM((CH * P, C), jnp.float32),
            pltpu.VMEM((CH * P, C), jnp.float32),
            pltpu.VMEM((P, 16), jnp.float32),
            pltpu.VMEM((P * 16,), jnp.float32),
            pltpu.SemaphoreType.DMA,
            pltpu.SemaphoreType.DMA,
            pltpu.SemaphoreType.DMA,
            pltpu.SemaphoreType.DMA,
        ],
    )(_sc_body)
    return fn(feat_flat, cal_flat, topk_flat, gamma_vec)


KB = 256
KT = K // KB


def _relayout_body(x_ref, o_ref):
    x = x_ref[...]
    o_ref[...] = jnp.transpose(x, (1, 0)).reshape(P, C, KB)


def _run_relayout(x):
    return pl.pallas_call(
        _relayout_body,
        grid=(B, KT),
        in_specs=[pl.BlockSpec((None, None, KB, P * C), lambda b, t: (b, t, 0, 0))],
        out_specs=pl.BlockSpec((None, P, C, KB), lambda b, t: (b, 0, 0, t)),
        out_shape=jax.ShapeDtypeStruct((B, P, C, K), jnp.float32),
    )(x)


def _offsets_const():
    y = jnp.arange(-PAD, PAD + 1)
    x = jnp.arange(-PAD, PAD + 1)
    gy, gx = jnp.meshgrid(y, x, indexing='ij')
    return jnp.stack([gy.flatten(), gx.flatten()], axis=-1).reshape(1, 1, P, 2)


@jax.jit
def kernel(feat_map, saliency_map, mask_logits, gamma):
    sal_flat = saliency_map.reshape(B, ROWS, 128)
    msk_flat = mask_logits.reshape(1, ROWS, 128)
    cal, idxp, rowp, colp = _run_topk(sal_flat, msk_flat)

    calibrated_map = cal.reshape(B, H, W)
    topk_idx = idxp.reshape(B, OUT_ROWS * 128)[:, :K]
    rows = rowp.reshape(B, OUT_ROWS * 128)[:, :K]
    cols = colp.reshape(B, OUT_ROWS * 128)[:, :K]
    topk_coords = jnp.stack([rows, cols], axis=-1)

    feat_flat = jnp.transpose(feat_map, (0, 2, 3, 1)).reshape(B * N, C)
    cal_flat = cal.reshape(B * N)
    gamma_vec = jnp.broadcast_to(gamma.reshape(1), (16,)).astype(jnp.float32)

    out = _run_sc(feat_flat, cal_flat, topk_idx.reshape(B * K), gamma_vec)
    bpck = _run_relayout(out.reshape(B, KT, KB, P * C))
    patches = jnp.transpose(bpck, (0, 3, 1, 2))

    return (patches, topk_coords, _offsets_const(), calibrated_map)
